# Initial kernel scaffold; baseline (speedup 1.0000x reference)
import functools

import jax
import jax.numpy as jnp
from jax import lax
from jax.experimental import pallas as pl
from jax.experimental.pallas import tpu as pltpu
from jax.experimental.pallas import tpu_sc as plsc

_N = 10000
_D = 128
_E = 320000
_EP = 100000
_NC, _NS = 2, 16
_NW = _NC * _NS
_K = 128
_CH = 79
_NACC = 10240
_RPT = _NACC // _NS
_CHP = 49
_EPPAD = (_NW // 2) * _CHP * _K
_PTOT = 2 * _EPPAD


def _mesh():
    return plsc.VectorSubcoreMesh(core_axis_name="c", subcore_axis_name="s")


def _agg_body(with_deg, *refs):
    if with_deg:
        (table_h, src_h, dst_h, z_h, zd_h, ones_h, part_h, degp_h,
         src_v, dst_v, rows_v, acc_s, onek_v, dega_s, sem) = refs
    else:
        (table_h, src_h, dst_h, z_h, part_h,
         src_v, dst_v, rows_v, acc_s, sem) = refs
    c = lax.axis_index("c")
    s = lax.axis_index("s")
    wid = s * _NC + c
    r0 = s * _RPT
    pltpu.sync_copy(z_h.at[pl.ds(r0, _RPT)], acc_s.at[pl.ds(r0, _RPT)])
    if with_deg:
        pltpu.sync_copy(zd_h.at[pl.ds(r0, _RPT)], dega_s.at[pl.ds(r0, _RPT)])
        pltpu.sync_copy(ones_h, onek_v)
    pltpu.sync_copy(src_h.at[wid], src_v)
    pltpu.sync_copy(dst_h.at[wid], dst_v)
    plsc.subcore_barrier()

    def step(j, carry):
        pltpu.async_copy(table_h.at[src_v.at[j]], rows_v, sem).wait()
        pltpu.sync_copy(rows_v, acc_s.at[dst_v.at[j]], add=True)
        if with_deg:
            pltpu.sync_copy(onek_v, dega_s.at[dst_v.at[j]], add=True)
        return carry

    lax.fori_loop(0, _CH, step, 0)
    plsc.subcore_barrier()
    pltpu.sync_copy(acc_s.at[pl.ds(r0, _RPT)], part_h.at[c, pl.ds(r0, _RPT)])
    if with_deg:
        pltpu.sync_copy(dega_s.at[pl.ds(r0, _RPT)], degp_h.at[c, pl.ds(r0, _RPT)])


def _sage_agg_deg(table, srcp, dstp, z, zd, ones_k):
    out_type = [
        jax.ShapeDtypeStruct((_NC, _NACC, _D), jnp.float32),
        jax.ShapeDtypeStruct((_NC, _NACC, 16), jnp.float32),
    ]
    scratch = [
        pltpu.VMEM((_CH, _K), jnp.int32),
        pltpu.VMEM((_CH, _K), jnp.int32),
        pltpu.VMEM((_K, _D), jnp.float32),
        pltpu.VMEM_SHARED((_NACC, _D), jnp.float32),
        pltpu.VMEM((_K, 16), jnp.float32),
        pltpu.VMEM_SHARED((_NACC, 16), jnp.float32),
        pltpu.SemaphoreType.DMA,
    ]
    f = pl.kernel(functools.partial(_agg_body, True), out_type=out_type,
                  mesh=_mesh(), scratch_types=scratch)
    return f(table, srcp, dstp, z, zd, ones_k)


def _sage_agg(table, srcp, dstp, z):
    out_type = jax.ShapeDtypeStruct((_NC, _NACC, _D), jnp.float32)
    scratch = [
        pltpu.VMEM((_CH, _K), jnp.int32),
        pltpu.VMEM((_CH, _K), jnp.int32),
        pltpu.VMEM((_K, _D), jnp.float32),
        pltpu.VMEM_SHARED((_NACC, _D), jnp.float32),
        pltpu.SemaphoreType.DMA,
    ]
    f = pl.kernel(functools.partial(_agg_body, False), out_type=out_type,
                  mesh=_mesh(), scratch_types=scratch)
    return f(table, srcp, dstp, z)


def _pair_gather_body(h_h, src_h, dst_h, a_h, b_h, src_v, dst_v, av, bv, sem):
    c = lax.axis_index("c")
    s = lax.axis_index("s")
    wid = s * _NC + c
    base = wid * _CHP
    pltpu.sync_copy(src_h.at[wid], src_v)
    pltpu.sync_copy(dst_h.at[wid], dst_v)

    def step(j, carry):
        pltpu.async_copy(h_h.at[src_v.at[j]], av, sem).wait()
        pltpu.async_copy(h_h.at[dst_v.at[j]], bv, sem).wait()
        row = (base + j) * _K
        pltpu.sync_copy(av, a_h.at[pl.ds(row, _K)])
        pltpu.sync_copy(bv, b_h.at[pl.ds(row, _K)])
        return carry

    lax.fori_loop(0, _CHP, step, 0)


def _pair_gather(h, srcp, dstp):
    out_type = [
        jax.ShapeDtypeStruct((_PTOT, _D), jnp.float32),
        jax.ShapeDtypeStruct((_PTOT, _D), jnp.float32),
    ]
    scratch = [
        pltpu.VMEM((_CHP, _K), jnp.int32),
        pltpu.VMEM((_CHP, _K), jnp.int32),
        pltpu.VMEM((_K, _D), jnp.float32),
        pltpu.VMEM((_K, _D), jnp.float32),
        pltpu.SemaphoreType.DMA,
    ]
    f = pl.kernel(_pair_gather_body, out_type=out_type, mesh=_mesh(),
                  scratch_types=scratch)
    return f(h, srcp, dstp)


def _layer_tc(h, parts, degp, Ws, Wn, b, relu):
    R = 1000

    def body(h_ref, p_ref, d_ref, ws_ref, wn_ref, b_ref, o_ref):
        deg = jnp.maximum(d_ref[0, :, 0] + d_ref[1, :, 0], 1.0)
        agg = (p_ref[0] + p_ref[1]) / deg[:, None]
        o = (jnp.dot(h_ref[...], ws_ref[...], preferred_element_type=jnp.float32)
             + jnp.dot(agg, wn_ref[...], preferred_element_type=jnp.float32)
             + b_ref[...])
        if relu:
            o = jnp.maximum(o, 0.0)
        o_ref[...] = o

    return pl.pallas_call(
        body,
        grid=(_N // R,),
        in_specs=[
            pl.BlockSpec((R, _D), lambda i: (i, 0)),
            pl.BlockSpec((_NC, R, _D), lambda i: (0, i, 0)),
            pl.BlockSpec((_NC, R, 16), lambda i: (0, i, 0)),
            pl.BlockSpec((_D, _D), lambda i: (0, 0)),
            pl.BlockSpec((_D, _D), lambda i: (0, 0)),
            pl.BlockSpec((1, _D), lambda i: (0, 0)),
        ],
        out_specs=pl.BlockSpec((R, _D), lambda i: (i, 0)),
        out_shape=jax.ShapeDtypeStruct((_N, _D), jnp.float32),
    )(h, parts, degp, Ws, Wn, b.reshape(1, _D))


def _mlp_tc(a, b, P0, pb0, P1, pb1, P2, pb2):
    R = 1024

    def body(a_ref, b_ref, p0, q0, p1, q1, p2, q2, o_ref):
        z = a_ref[...] * b_ref[...]
        z = jnp.maximum(
            jnp.dot(z, p0[...], preferred_element_type=jnp.float32) + q0[...], 0.0)
        z = jnp.maximum(
            jnp.dot(z, p1[...], preferred_element_type=jnp.float32) + q1[...], 0.0)
        o_ref[...] = jnp.dot(z, p2[...], preferred_element_type=jnp.float32) + q2[...]

    return pl.pallas_call(
        body,
        grid=(_PTOT // R,),
        in_specs=[
            pl.BlockSpec((R, _D), lambda i: (i, 0)),
            pl.BlockSpec((R, _D), lambda i: (i, 0)),
            pl.BlockSpec((_D, _D), lambda i: (0, 0)),
            pl.BlockSpec((1, _D), lambda i: (0, 0)),
            pl.BlockSpec((_D, _D), lambda i: (0, 0)),
            pl.BlockSpec((1, _D), lambda i: (0, 0)),
            pl.BlockSpec((_D, 1), lambda i: (0, 0)),
            pl.BlockSpec((1, 1), lambda i: (0, 0)),
        ],
        out_specs=pl.BlockSpec((R, 1), lambda i: (i, 0)),
        out_shape=jax.ShapeDtypeStruct((_PTOT, 1), jnp.float32),
    )(a, b, P0, pb0.reshape(1, _D), P1, pb1.reshape(1, _D), P2,
      pb2.reshape(1, 1))


def kernel(x, edge_index, pos_src, pos_dst, neg_src, neg_dst,
           W_self0, W_neigh0, b0, W_self1, W_neigh1, b1, W_self2, W_neigh2, b2,
           P0, pb0, P1, pb1, P2, pb2):
    src = edge_index[0]
    dst = edge_index[1]
    pad_e = _NW * _CH * _K - _E
    srcp = jnp.concatenate(
        [src, jnp.zeros((pad_e,), jnp.int32)]).reshape(_NW, _CH, _K)
    dstp = jnp.concatenate(
        [dst, jnp.full((pad_e,), _N, jnp.int32)]).reshape(_NW, _CH, _K)
    z = jnp.zeros((_NACC, _D), jnp.float32)
    zd = jnp.zeros((_NACC, 16), jnp.float32)
    ones_k = jnp.ones((_K, 16), jnp.float32)

    parts, degp = _sage_agg_deg(x, srcp, dstp, z, zd, ones_k)
    h1 = _layer_tc(x, parts, degp, W_self0, W_neigh0, b0, relu=True)
    parts = _sage_agg(h1, srcp, dstp, z)
    h2 = _layer_tc(h1, parts, degp, W_self1, W_neigh1, b1, relu=True)
    parts = _sage_agg(h2, srcp, dstp, z)
    h3 = _layer_tc(h2, parts, degp, W_self2, W_neigh2, b2, relu=False)

    pad_p = _EPPAD - _EP
    zp = jnp.zeros((pad_p,), jnp.int32)
    ps = jnp.concatenate([pos_src, zp, neg_src, zp]).reshape(_NW, _CHP, _K)
    pd = jnp.concatenate([pos_dst, zp, neg_dst, zp]).reshape(_NW, _CHP, _K)
    a, bm = _pair_gather(h3, ps, pd)
    scores = _mlp_tc(a, bm, P0, pb0, P1, pb1, P2, pb2)
    return scores[:_EP], scores[_EPPAD:_EPPAD + _EP]


# speedup vs baseline: 2.3329x; 2.3329x over previous
import jax
import jax.numpy as jnp
from jax import lax
from jax.experimental import pallas as pl
from jax.experimental.pallas import tpu as pltpu
from jax.experimental.pallas import tpu_sc as plsc

_N = 10000
_D = 128
_E = 320000
_EP = 100000
_NC, _NS = 2, 16
_NW = _NC * _NS
_K = 128
_GK = 8
_G = 10
_CH = _G * _GK
_NACC = 10112
_RPT = _NACC // _NS
_CS = (128, 128, 128, 128, 120)
_CHP = 49
_EPPAD = (_NW // 2) * _CHP * _K
_PTOT = 2 * _EPPAD


def _mesh():
    return plsc.VectorSubcoreMesh(core_axis_name="c", subcore_axis_name="s")


def _agg_body(table_h, src_h, dst_h, z_h, part_h,
              src_v, dst_v, rows_v, acc_s, sem):
    c = lax.axis_index("c")
    s = lax.axis_index("s")
    wid = s * _NC + c
    r0 = s * _RPT
    pltpu.sync_copy(z_h, rows_v)
    off = 0
    for n in _CS:
        pltpu.sync_copy(rows_v.at[pl.ds(0, n)], acc_s.at[pl.ds(r0 + off, n)])
        off += n
    plsc.subcore_barrier()

    def group(g, carry):
        pltpu.sync_copy(src_h.at[wid, pl.ds(g * _GK, _GK)], src_v)
        pltpu.sync_copy(dst_h.at[wid, pl.ds(g * _GK, _GK)], dst_v)

        def step(j, c2):
            pltpu.async_copy(table_h.at[src_v.at[j]], rows_v, sem).wait()
            pltpu.sync_copy(rows_v, acc_s.at[dst_v.at[j]], add=True)
            return c2

        lax.fori_loop(0, _GK, step, 0)
        return carry

    lax.fori_loop(0, _G, group, 0)
    plsc.subcore_barrier()
    off = 0
    for n in _CS:
        pltpu.sync_copy(acc_s.at[pl.ds(r0 + off, n)], rows_v.at[pl.ds(0, n)])
        pltpu.sync_copy(rows_v.at[pl.ds(0, n)], part_h.at[c, pl.ds(r0 + off, n)])
        off += n


def _sage_agg(table, srcp, dstp, z):
    out_type = jax.ShapeDtypeStruct((_NC, _NACC, _D), jnp.float32)
    scratch = [
        pltpu.VMEM((_GK, _K), jnp.int32),
        pltpu.VMEM((_GK, _K), jnp.int32),
        pltpu.VMEM((_K, _D), jnp.float32),
        pltpu.VMEM_SHARED((_NACC, _D), jnp.float32),
        pltpu.SemaphoreType.DMA,
    ]
    f = pl.kernel(_agg_body, out_type=out_type, mesh=_mesh(),
                  scratch_types=scratch)
    return f(table, srcp, dstp, z)


def _pair_gather_body(h_h, src_h, dst_h, a_h, b_h, src_v, dst_v, av, bv, sem):
    c = lax.axis_index("c")
    s = lax.axis_index("s")
    wid = s * _NC + c
    base = wid * _CHP
    pltpu.sync_copy(src_h.at[wid], src_v)
    pltpu.sync_copy(dst_h.at[wid], dst_v)

    def step(j, carry):
        pltpu.async_copy(h_h.at[src_v.at[j]], av, sem).wait()
        pltpu.async_copy(h_h.at[dst_v.at[j]], bv, sem).wait()
        row = (base + j) * _K
        pltpu.sync_copy(av, a_h.at[pl.ds(row, _K)])
        pltpu.sync_copy(bv, b_h.at[pl.ds(row, _K)])
        return carry

    lax.fori_loop(0, _CHP, step, 0)


def _pair_gather(h, srcp, dstp):
    out_type = [
        jax.ShapeDtypeStruct((_PTOT, _D), jnp.float32),
        jax.ShapeDtypeStruct((_PTOT, _D), jnp.float32),
    ]
    scratch = [
        pltpu.VMEM((_CHP, _K), jnp.int32),
        pltpu.VMEM((_CHP, _K), jnp.int32),
        pltpu.VMEM((_K, _D), jnp.float32),
        pltpu.VMEM((_K, _D), jnp.float32),
        pltpu.SemaphoreType.DMA,
    ]
    f = pl.kernel(_pair_gather_body, out_type=out_type, mesh=_mesh(),
                  scratch_types=scratch)
    return f(h, srcp, dstp)


def _layer_tc(h, parts, degp, Ws, Wn, b, relu):
    R = 1000

    def body(h_ref, p_ref, d_ref, ws_ref, wn_ref, b_ref, o_ref):
        deg = jnp.maximum(d_ref[0, :, 0] + d_ref[1, :, 0], 1.0)
        agg = (p_ref[0] + p_ref[1]) / deg[:, None]
        o = (jnp.dot(h_ref[...], ws_ref[...], preferred_element_type=jnp.float32)
             + jnp.dot(agg, wn_ref[...], preferred_element_type=jnp.float32)
             + b_ref[...])
        if relu:
            o = jnp.maximum(o, 0.0)
        o_ref[...] = o

    return pl.pallas_call(
        body,
        grid=(_N // R,),
        in_specs=[
            pl.BlockSpec((R, _D), lambda i: (i, 0)),
            pl.BlockSpec((_NC, R, _D), lambda i: (0, i, 0)),
            pl.BlockSpec((_NC, R, _D), lambda i: (0, i, 0)),
            pl.BlockSpec((_D, _D), lambda i: (0, 0)),
            pl.BlockSpec((_D, _D), lambda i: (0, 0)),
            pl.BlockSpec((1, _D), lambda i: (0, 0)),
        ],
        out_specs=pl.BlockSpec((R, _D), lambda i: (i, 0)),
        out_shape=jax.ShapeDtypeStruct((_N, _D), jnp.float32),
    )(h, parts, degp, Ws, Wn, b.reshape(1, _D))


def _mlp_tc(a, b, P0, pb0, P1, pb1, P2, pb2):
    R = 1024

    def body(a_ref, b_ref, p0, q0, p1, q1, p2, q2, o_ref):
        z = a_ref[...] * b_ref[...]
        z = jnp.maximum(
            jnp.dot(z, p0[...], preferred_element_type=jnp.float32) + q0[...], 0.0)
        z = jnp.maximum(
            jnp.dot(z, p1[...], preferred_element_type=jnp.float32) + q1[...], 0.0)
        o_ref[...] = jnp.dot(z, p2[...], preferred_element_type=jnp.float32) + q2[...]

    return pl.pallas_call(
        body,
        grid=(_PTOT // R,),
        in_specs=[
            pl.BlockSpec((R, _D), lambda i: (i, 0)),
            pl.BlockSpec((R, _D), lambda i: (i, 0)),
            pl.BlockSpec((_D, _D), lambda i: (0, 0)),
            pl.BlockSpec((1, _D), lambda i: (0, 0)),
            pl.BlockSpec((_D, _D), lambda i: (0, 0)),
            pl.BlockSpec((1, _D), lambda i: (0, 0)),
            pl.BlockSpec((_D, 1), lambda i: (0, 0)),
            pl.BlockSpec((1, 1), lambda i: (0, 0)),
        ],
        out_specs=pl.BlockSpec((R, 1), lambda i: (i, 0)),
        out_shape=jax.ShapeDtypeStruct((_PTOT, 1), jnp.float32),
    )(a, b, P0, pb0.reshape(1, _D), P1, pb1.reshape(1, _D), P2,
      pb2.reshape(1, 1))


def kernel(x, edge_index, pos_src, pos_dst, neg_src, neg_dst,
           W_self0, W_neigh0, b0, W_self1, W_neigh1, b1, W_self2, W_neigh2, b2,
           P0, pb0, P1, pb1, P2, pb2):
    src = edge_index[0]
    dst = edge_index[1]
    pad_e = _NW * _CH * _K - _E
    srcp = jnp.concatenate(
        [src, jnp.zeros((pad_e,), jnp.int32)]).reshape(_NW, _CH, _K)
    dstp = jnp.concatenate(
        [dst, jnp.full((pad_e,), _N, jnp.int32)]).reshape(_NW, _CH, _K)
    z = jnp.zeros((_K, _D), jnp.float32)

    ones_table = jnp.ones((_K, _D), jnp.float32)
    src_iota = jnp.broadcast_to(
        jnp.arange(_K, dtype=jnp.int32), (_NW, _CH, _K))
    degp = _sage_agg(ones_table, src_iota, dstp, z)

    parts = _sage_agg(x, srcp, dstp, z)
    h1 = _layer_tc(x, parts, degp, W_self0, W_neigh0, b0, relu=True)
    parts = _sage_agg(h1, srcp, dstp, z)
    h2 = _layer_tc(h1, parts, degp, W_self1, W_neigh1, b1, relu=True)
    parts = _sage_agg(h2, srcp, dstp, z)
    h3 = _layer_tc(h2, parts, degp, W_self2, W_neigh2, b2, relu=False)

    pad_p = _EPPAD - _EP
    zp = jnp.zeros((pad_p,), jnp.int32)
    ps = jnp.concatenate([pos_src, zp, neg_src, zp]).reshape(_NW, _CHP, _K)
    pd = jnp.concatenate([pos_dst, zp, neg_dst, zp]).reshape(_NW, _CHP, _K)
    a, bm = _pair_gather(h3, ps, pd)
    scores = _mlp_tc(a, bm, P0, pb0, P1, pb1, P2, pb2)
    return scores[:_EP], scores[_EPPAD:_EPPAD + _EP]


# speedup vs baseline: 2.5641x; 1.0991x over previous
import jax
import jax.numpy as jnp
from jax import lax
from jax.experimental import pallas as pl
from jax.experimental.pallas import tpu as pltpu
from jax.experimental.pallas import tpu_sc as plsc

_N = 10000
_D = 128
_E = 320000
_EP = 100000
_NC, _NS = 2, 16
_NW = _NC * _NS
_K = 128
_GK = 16
_G = 5
_CH = _G * _GK
_NACC = 10112
_RPT = _NACC // _NS
_CS = (128, 128, 128, 128, 120)
_CHP = 49
_EPPAD = (_NW // 2) * _CHP * _K
_PTOT = 2 * _EPPAD


def _mesh():
    return plsc.VectorSubcoreMesh(core_axis_name="c", subcore_axis_name="s")


def _agg_body(table_h, src_h, dst_h, z_h, part_h,
              src_v, dst_v, rows0, rows1, acc_s,
              gsem0, gsem1, ssem0, ssem1):
    rows = (rows0, rows1)
    gsem = (gsem0, gsem1)
    ssem = (ssem0, ssem1)
    c = lax.axis_index("c")
    s = lax.axis_index("s")
    wid = s * _NC + c
    r0 = s * _RPT
    pltpu.sync_copy(z_h, rows0)
    off = 0
    for n in _CS:
        pltpu.sync_copy(rows0.at[pl.ds(0, n)], acc_s.at[pl.ds(r0 + off, n)])
        off += n
    plsc.subcore_barrier()

    def group(g, carry):
        pltpu.sync_copy(src_h.at[wid, pl.ds(g * _GK, _GK)], src_v)
        pltpu.sync_copy(dst_h.at[wid, pl.ds(g * _GK, _GK)], dst_v)
        gcp = [None] * _GK
        scp = [None] * _GK
        for t in range(_GK):
            b = t % 2
            if t >= 2:
                scp[t - 2].wait()
            gcp[t] = pltpu.async_copy(
                table_h.at[src_v.at[t]], rows[b], gsem[b])
            if t >= 1:
                p = (t - 1) % 2
                gcp[t - 1].wait()
                scp[t - 1] = pltpu.async_copy(
                    rows[p], acc_s.at[dst_v.at[t - 1]], ssem[p], add=True)
        last = _GK - 1
        gcp[last].wait()
        scp[last] = pltpu.async_copy(
            rows[last % 2], acc_s.at[dst_v.at[last]], ssem[last % 2], add=True)
        scp[last - 1].wait()
        scp[last].wait()
        return carry

    lax.fori_loop(0, _G, group, 0)
    plsc.subcore_barrier()
    off = 0
    for n in _CS:
        pltpu.sync_copy(acc_s.at[pl.ds(r0 + off, n)], rows0.at[pl.ds(0, n)])
        pltpu.sync_copy(rows0.at[pl.ds(0, n)], part_h.at[c, pl.ds(r0 + off, n)])
        off += n


def _sage_agg(table, srcp, dstp, z):
    out_type = jax.ShapeDtypeStruct((_NC, _NACC, _D), jnp.float32)
    scratch = [
        pltpu.VMEM((_GK, _K), jnp.int32),
        pltpu.VMEM((_GK, _K), jnp.int32),
        pltpu.VMEM((_K, _D), jnp.float32),
        pltpu.VMEM((_K, _D), jnp.float32),
        pltpu.VMEM_SHARED((_NACC, _D), jnp.float32),
        pltpu.SemaphoreType.DMA,
        pltpu.SemaphoreType.DMA,
        pltpu.SemaphoreType.DMA,
        pltpu.SemaphoreType.DMA,
    ]
    f = pl.kernel(_agg_body, out_type=out_type, mesh=_mesh(),
                  scratch_types=scratch)
    return f(table, srcp, dstp, z)


_PGK = 7


def _pair_gather_body(h_h, src_h, dst_h, a_h, b_h, src_v, dst_v,
                      av0, av1, bv0, bv1,
                      gsa0, gsa1, gsb0, gsb1, wsa0, wsa1, wsb0, wsb1):
    av = (av0, av1)
    bv = (bv0, bv1)
    gsa = (gsa0, gsa1)
    gsb = (gsb0, gsb1)
    wsa = (wsa0, wsa1)
    wsb = (wsb0, wsb1)
    c = lax.axis_index("c")
    s = lax.axis_index("s")
    wid = s * _NC + c
    base = wid * _CHP
    pltpu.sync_copy(src_h.at[wid], src_v)
    pltpu.sync_copy(dst_h.at[wid], dst_v)

    def group(g, carry):
        j0 = g * _PGK
        ga = [None] * _PGK
        gb = [None] * _PGK
        wa = [None] * _PGK
        wb = [None] * _PGK
        for t in range(_PGK):
            b = t % 2
            if t >= 2:
                wa[t - 2].wait()
                wb[t - 2].wait()
            ga[t] = pltpu.async_copy(h_h.at[src_v.at[j0 + t]], av[b], gsa[b])
            gb[t] = pltpu.async_copy(h_h.at[dst_v.at[j0 + t]], bv[b], gsb[b])
            if t >= 1:
                p = (t - 1) % 2
                row = (base + j0 + t - 1) * _K
                ga[t - 1].wait()
                wa[t - 1] = pltpu.async_copy(
                    av[p], a_h.at[pl.ds(row, _K)], wsa[p])
                gb[t - 1].wait()
                wb[t - 1] = pltpu.async_copy(
                    bv[p], b_h.at[pl.ds(row, _K)], wsb[p])
        last = _PGK - 1
        p = last % 2
        row = (base + j0 + last) * _K
        ga[last].wait()
        wa[last] = pltpu.async_copy(av[p], a_h.at[pl.ds(row, _K)], wsa[p])
        gb[last].wait()
        wb[last] = pltpu.async_copy(bv[p], b_h.at[pl.ds(row, _K)], wsb[p])
        wa[last - 1].wait()
        wb[last - 1].wait()
        wa[last].wait()
        wb[last].wait()
        return carry

    lax.fori_loop(0, _CHP // _PGK, group, 0)


def _pair_gather(h, srcp, dstp):
    out_type = [
        jax.ShapeDtypeStruct((_PTOT, _D), jnp.float32),
        jax.ShapeDtypeStruct((_PTOT, _D), jnp.float32),
    ]
    scratch = [
        pltpu.VMEM((_CHP, _K), jnp.int32),
        pltpu.VMEM((_CHP, _K), jnp.int32),
        pltpu.VMEM((_K, _D), jnp.float32),
        pltpu.VMEM((_K, _D), jnp.float32),
        pltpu.VMEM((_K, _D), jnp.float32),
        pltpu.VMEM((_K, _D), jnp.float32),
    ] + [pltpu.SemaphoreType.DMA] * 8
    f = pl.kernel(_pair_gather_body, out_type=out_type, mesh=_mesh(),
                  scratch_types=scratch)
    return f(h, srcp, dstp)


def _layer_tc(h, parts, degp, Ws, Wn, b, relu):
    R = 1000

    def body(h_ref, p_ref, d_ref, ws_ref, wn_ref, b_ref, o_ref):
        deg = jnp.maximum(d_ref[0, :, 0] + d_ref[1, :, 0], 1.0)
        agg = (p_ref[0] + p_ref[1]) / deg[:, None]
        o = (jnp.dot(h_ref[...], ws_ref[...], preferred_element_type=jnp.float32)
             + jnp.dot(agg, wn_ref[...], preferred_element_type=jnp.float32)
             + b_ref[...])
        if relu:
            o = jnp.maximum(o, 0.0)
        o_ref[...] = o

    return pl.pallas_call(
        body,
        grid=(_N // R,),
        in_specs=[
            pl.BlockSpec((R, _D), lambda i: (i, 0)),
            pl.BlockSpec((_NC, R, _D), lambda i: (0, i, 0)),
            pl.BlockSpec((_NC, R, _D), lambda i: (0, i, 0)),
            pl.BlockSpec((_D, _D), lambda i: (0, 0)),
            pl.BlockSpec((_D, _D), lambda i: (0, 0)),
            pl.BlockSpec((1, _D), lambda i: (0, 0)),
        ],
        out_specs=pl.BlockSpec((R, _D), lambda i: (i, 0)),
        out_shape=jax.ShapeDtypeStruct((_N, _D), jnp.float32),
    )(h, parts, degp, Ws, Wn, b.reshape(1, _D))


def _mlp_tc(a, b, P0, pb0, P1, pb1, P2, pb2):
    R = 1024

    def body(a_ref, b_ref, p0, q0, p1, q1, p2, q2, o_ref):
        z = a_ref[...] * b_ref[...]
        z = jnp.maximum(
            jnp.dot(z, p0[...], preferred_element_type=jnp.float32) + q0[...], 0.0)
        z = jnp.maximum(
            jnp.dot(z, p1[...], preferred_element_type=jnp.float32) + q1[...], 0.0)
        o_ref[...] = jnp.dot(z, p2[...], preferred_element_type=jnp.float32) + q2[...]

    return pl.pallas_call(
        body,
        grid=(_PTOT // R,),
        in_specs=[
            pl.BlockSpec((R, _D), lambda i: (i, 0)),
            pl.BlockSpec((R, _D), lambda i: (i, 0)),
            pl.BlockSpec((_D, _D), lambda i: (0, 0)),
            pl.BlockSpec((1, _D), lambda i: (0, 0)),
            pl.BlockSpec((_D, _D), lambda i: (0, 0)),
            pl.BlockSpec((1, _D), lambda i: (0, 0)),
            pl.BlockSpec((_D, 1), lambda i: (0, 0)),
            pl.BlockSpec((1, 1), lambda i: (0, 0)),
        ],
        out_specs=pl.BlockSpec((R, 1), lambda i: (i, 0)),
        out_shape=jax.ShapeDtypeStruct((_PTOT, 1), jnp.float32),
    )(a, b, P0, pb0.reshape(1, _D), P1, pb1.reshape(1, _D), P2,
      pb2.reshape(1, 1))


def kernel(x, edge_index, pos_src, pos_dst, neg_src, neg_dst,
           W_self0, W_neigh0, b0, W_self1, W_neigh1, b1, W_self2, W_neigh2, b2,
           P0, pb0, P1, pb1, P2, pb2):
    src = edge_index[0]
    dst = edge_index[1]
    pad_e = _NW * _CH * _K - _E
    srcp = jnp.concatenate(
        [src, jnp.zeros((pad_e,), jnp.int32)]).reshape(_NW, _CH, _K)
    dstp = jnp.concatenate(
        [dst, jnp.full((pad_e,), _N, jnp.int32)]).reshape(_NW, _CH, _K)
    z = jnp.zeros((_K, _D), jnp.float32)

    ones_table = jnp.ones((_K, _D), jnp.float32)
    src_iota = jnp.broadcast_to(
        jnp.arange(_K, dtype=jnp.int32), (_NW, _CH, _K))
    degp = _sage_agg(ones_table, src_iota, dstp, z)

    parts = _sage_agg(x, srcp, dstp, z)
    h1 = _layer_tc(x, parts, degp, W_self0, W_neigh0, b0, relu=True)
    parts = _sage_agg(h1, srcp, dstp, z)
    h2 = _layer_tc(h1, parts, degp, W_self1, W_neigh1, b1, relu=True)
    parts = _sage_agg(h2, srcp, dstp, z)
    h3 = _layer_tc(h2, parts, degp, W_self2, W_neigh2, b2, relu=False)

    pad_p = _EPPAD - _EP
    zp = jnp.zeros((pad_p,), jnp.int32)
    ps = jnp.concatenate([pos_src, zp, neg_src, zp]).reshape(_NW, _CHP, _K)
    pd = jnp.concatenate([pos_dst, zp, neg_dst, zp]).reshape(_NW, _CHP, _K)
    a, bm = _pair_gather(h3, ps, pd)
    scores = _mlp_tc(a, bm, P0, pb0, P1, pb1, P2, pb2)
    return scores[:_EP], scores[_EPPAD:_EPPAD + _EP]
